# Initial kernel scaffold; baseline (speedup 1.0000x reference)
import jax
import jax.numpy as jnp
from jax.experimental import pallas as pl


def kernel(idx, directions):
    raise NotImplementedError("write your pallas kernel here")


# speedup vs baseline: 2.7698x; 2.7698x over previous
import jax
import jax.numpy as jnp
from jax.experimental import pallas as pl
from jax.experimental.pallas import tpu as pltpu
from jax.experimental.pallas import tpu_sc as plsc

_ENC = 32
_W = 128


def kernel(idx, directions):
    b, s = idx.shape
    n = b * s
    assert n % _W == 0
    idx_flat = idx.reshape(1, n)
    mesh = plsc.VectorSubcoreMesh(core_axis_name="core", subcore_axis_name="subcore")

    @pl.kernel(out_type=jax.ShapeDtypeStruct((n, _ENC), directions.dtype),
               mesh=mesh,
               compiler_params=pltpu.CompilerParams(use_tc_tiling_on_sc=False))
    def gather_kernel(dirs_hbm, idx_hbm, out_hbm):
        def body(i_vmem, o_vmem):
            pltpu.sync_copy(dirs_hbm.at[i_vmem.at[0]], o_vmem)

        pltpu.emit_pipeline(
            body,
            grid=(n // _W,),
            in_specs=[pl.BlockSpec((1, _W), index_map=lambda i: (0, i))],
            out_specs=[pl.BlockSpec((_W, _ENC), index_map=lambda i: (i, 0))],
            core_axis_name=("core", "subcore"),
            dimension_semantics=(pltpu.PARALLEL,),
        )(idx_hbm, out_hbm)

    return gather_kernel(directions, idx_flat).reshape(b, s, _ENC)


# speedup vs baseline: 3.0033x; 1.0843x over previous
import jax
import jax.numpy as jnp
from jax.experimental import pallas as pl
from jax.experimental.pallas import tpu as pltpu
from jax.experimental.pallas import tpu_sc as plsc

_ENC = 32
_W = 128
_K = 8
_BLK = _W * _K


def kernel(idx, directions):
    b, s = idx.shape
    n = b * s
    assert n % _BLK == 0
    idx_flat = idx.reshape(1, n)
    mesh = plsc.VectorSubcoreMesh(core_axis_name="core", subcore_axis_name="subcore")

    @pl.kernel(out_type=jax.ShapeDtypeStruct((n, _ENC), directions.dtype),
               mesh=mesh,
               scratch_types=[pltpu.SemaphoreType.DMA],
               compiler_params=pltpu.CompilerParams(use_tc_tiling_on_sc=False))
    def gather_kernel(dirs_hbm, idx_hbm, out_hbm, gsem):
        def body(i_vmem, o_vmem):
            handles = [
                pltpu.async_copy(
                    dirs_hbm.at[i_vmem.at[0, pl.ds(j * _W, _W)]],
                    o_vmem.at[pl.ds(j * _W, _W), :],
                    gsem,
                )
                for j in range(_K)
            ]
            for h in handles:
                h.wait()

        pltpu.emit_pipeline(
            body,
            grid=(n // _BLK,),
            in_specs=[pl.BlockSpec((1, _BLK), index_map=lambda i: (0, i))],
            out_specs=[pl.BlockSpec((_BLK, _ENC), index_map=lambda i: (i, 0))],
            core_axis_name=("core", "subcore"),
            dimension_semantics=(pltpu.PARALLEL,),
        )(idx_hbm, out_hbm)

    return gather_kernel(directions, idx_flat).reshape(b, s, _ENC)
